# baseline (speedup 1.0000x reference)
import functools

import jax
import jax.numpy as jnp
import numpy as np
from jax.experimental import pallas as pl
from jax.experimental.pallas import tpu as pltpu

MEM = 256
K = 4
SMK = 32
HID = 1024
INP = 1024
B = 8192
EPS = 1e-12
BLK = 512

NUP = 256
_U0A, _U1A, _U0B, _U1B, _A0, _B0 = 0, SMK, 2 * SMK, 3 * SMK, 4 * SMK, 4 * SMK + K


def _build_consts():
    ms = np.zeros((128, 16), np.float32)
    for i in range(SMK):
        ms[_U0A + i, i // 8] = 1.0
        ms[_U0B + i, 4 + i // 8] = 1.0
        ms[_U1A + i, 8] = 1.0
        ms[_U1B + i, 9] = 1.0
    mn0 = np.zeros((16, 8), np.float32)
    mn1 = np.zeros((16, 8), np.float32)
    for k in range(K):
        mn0[k, k] = 1.0
        mn0[4 + k, 4 + k] = 1.0
        mn1[8, k] = 1.0
        mn1[9, 4 + k] = 1.0
    mcexp = np.zeros((8, 64), np.float32)
    for i in range(SMK):
        mcexp[i // 8, i] = 1.0
        mcexp[4 + i // 8, 32 + i] = 1.0
    mcomb = np.zeros((128, 640), np.float32)
    for i in range(SMK):
        mcomb[_U0A + i, i] = 1.0
        mcomb[_U0B + i, 32 + i] = 1.0
    for m in range(MEM):
        mcomb[_U1A + m % 32, 128 + m] = 1.0
        mcomb[_U1B + m % 32, 384 + m] = 1.0
    mfold = np.zeros((64, 16), np.float32)
    for i in range(SMK):
        mfold[i, i % 8] = 1.0
        mfold[32 + i, 8 + i % 8] = 1.0
    mgexp = np.zeros((16, 512), np.float32)
    for m in range(MEM):
        mgexp[m // 32, m] = 1.0
        mgexp[8 + m // 32, 256 + m] = 1.0
    return ms, mn0, mn1, mcexp, mcomb, mfold, mgexp


_CONSTS = _build_consts()


def _nru_kernel(x_ref, h0_ref, mem_ref, wh_ref, bh_ref,
                wct_ref, bc_ref,
                ms_ref, mn0_ref, mn1_ref, mcexp_ref, mcomb_ref,
                mfold_ref, mgexp_ref,
                memnew_ref, h_ref, wbf_ref):
    f32 = jnp.float32
    bf16 = jnp.bfloat16
    dnn = (((1,), (0,)), ((), ()))

    @pl.when(pl.program_id(0) == 0)
    def _cast_weights():
        wbf_ref[...] = wh_ref[...].T.astype(bf16)

    mem = mem_ref[...]
    memb = mem.astype(bf16)
    cin = jnp.concatenate(
        [x_ref[...].astype(bf16), h0_ref[...].astype(bf16), memb], axis=1)
    acc = jax.lax.dot_general(cin, wbf_ref[...], dnn, preferred_element_type=f32)
    h = jnp.maximum(acc + bh_ref[...], 0.0)
    h_ref[...] = h

    hm = jnp.concatenate([h.astype(bf16), memb], axis=1)
    u = jax.lax.dot_general(hm, wct_ref[...], dnn, preferred_element_type=f32)
    u += bc_ref[...]

    uv = u[:, :128]
    ab = u[:, _A0:_A0 + 2 * K]
    u2 = uv * uv
    p = u2 * u2 * jnp.abs(uv)
    s = jnp.dot(p, ms_ref[...], preferred_element_type=f32)
    n5 = jnp.dot(s, mn0_ref[...], preferred_element_type=f32) \
        * jnp.dot(s, mn1_ref[...], preferred_element_type=f32)
    n = jnp.exp2(0.2 * jnp.log2(n5))
    coef = ab * (0.25 / jnp.maximum(n, EPS))
    sel = jnp.dot(uv, mcomb_ref[...], preferred_element_type=f32)
    cexp = jnp.dot(coef, mcexp_ref[...], preferred_element_type=f32)
    g = cexp * sel[:, :64]
    gf = jnp.dot(g, mfold_ref[...], preferred_element_type=f32)
    gexp = jnp.dot(gf, mgexp_ref[...], preferred_element_type=f32)
    prod = gexp * sel[:, 128:640]
    memnew_ref[...] = mem + (prod[:, :MEM] - prod[:, MEM:])


def kernel(x, h0, memory, W_h, b_h, W_a, b_a, W_b, b_b, W_va, b_va, W_vb, b_vb):
    zpad_w = jnp.zeros((NUP - 4 * SMK - 2 * K, HID + MEM), jnp.float32)
    wct = jnp.concatenate([W_va, W_vb, W_a, W_b, zpad_w],
                          axis=0).T.astype(jnp.bfloat16)
    zpad_b = jnp.zeros((NUP - 4 * SMK - 2 * K,), jnp.float32)
    bc = jnp.concatenate([b_va, b_vb, b_a, b_b, zpad_b])[None, :]
    bh = b_h[None, :]

    consts = [jnp.asarray(c) for c in _CONSTS]

    grid = (B // BLK,)
    row_spec = lambda cols: pl.BlockSpec((BLK, cols), lambda i: (i, 0))
    full = lambda a: pl.BlockSpec(a.shape, lambda i: (0,) * a.ndim)

    memnew, h = pl.pallas_call(
        _nru_kernel,
        grid=grid,
        in_specs=[row_spec(INP), row_spec(HID), row_spec(MEM),
                  full(W_h), full(bh),
                  full(wct), full(bc)] + [full(c) for c in consts],
        out_specs=[row_spec(MEM), row_spec(HID)],
        out_shape=[jax.ShapeDtypeStruct((B, MEM), jnp.float32),
                   jax.ShapeDtypeStruct((B, HID), jnp.float32)],
        scratch_shapes=[pltpu.VMEM((INP + HID + MEM, HID), jnp.bfloat16)],
        compiler_params=pltpu.CompilerParams(
            dimension_semantics=("arbitrary",),
            vmem_limit_bytes=100 * 1024 * 1024,
        ),
    )(x, h0, memory, W_h, bh, wct, bc, *consts)
    return memnew, h


# speedup vs baseline: 1.0255x; 1.0255x over previous
import functools

import jax
import jax.numpy as jnp
import numpy as np
from jax.experimental import pallas as pl
from jax.experimental.pallas import tpu as pltpu

MEM = 256
K = 4
SMK = 32
HID = 1024
INP = 1024
B = 8192
EPS = 1e-12
BLK = 1024

NUP = 256
_U0A, _U1A, _U0B, _U1B, _A0, _B0 = 0, SMK, 2 * SMK, 3 * SMK, 4 * SMK, 4 * SMK + K


def _build_consts():
    ms = np.zeros((128, 16), np.float32)
    for i in range(SMK):
        ms[_U0A + i, i // 8] = 1.0
        ms[_U0B + i, 4 + i // 8] = 1.0
        ms[_U1A + i, 8] = 1.0
        ms[_U1B + i, 9] = 1.0
    mn0 = np.zeros((16, 8), np.float32)
    mn1 = np.zeros((16, 8), np.float32)
    for k in range(K):
        mn0[k, k] = 1.0
        mn0[4 + k, 4 + k] = 1.0
        mn1[8, k] = 1.0
        mn1[9, 4 + k] = 1.0
    mcexp = np.zeros((8, 64), np.float32)
    for i in range(SMK):
        mcexp[i // 8, i] = 1.0
        mcexp[4 + i // 8, 32 + i] = 1.0
    mcomb = np.zeros((128, 640), np.float32)
    for i in range(SMK):
        mcomb[_U0A + i, i] = 1.0
        mcomb[_U0B + i, 32 + i] = 1.0
    for m in range(MEM):
        mcomb[_U1A + m % 32, 128 + m] = 1.0
        mcomb[_U1B + m % 32, 384 + m] = 1.0
    mfold = np.zeros((64, 16), np.float32)
    for i in range(SMK):
        mfold[i, i % 8] = 1.0
        mfold[32 + i, 8 + i % 8] = 1.0
    mgexp = np.zeros((16, 512), np.float32)
    for m in range(MEM):
        mgexp[m // 32, m] = 1.0
        mgexp[8 + m // 32, 256 + m] = 1.0
    return ms, mn0, mn1, mcexp, mcomb, mfold, mgexp


_CONSTS = _build_consts()


def _nru_kernel(x_ref, h0_ref, mem_ref, wh_ref, bh_ref,
                wct_ref, bc_ref,
                ms_ref, mn0_ref, mn1_ref, mcexp_ref, mcomb_ref,
                mfold_ref, mgexp_ref,
                memnew_ref, h_ref, wbf_ref, wraw_ref, wsem):
    f32 = jnp.float32
    bf16 = jnp.bfloat16
    dnn = (((1,), (0,)), ((), ()))

    @pl.when(pl.program_id(0) == 0)
    def _cast_weights():
        cp = pltpu.make_async_copy(wh_ref, wraw_ref, wsem)
        cp.start()
        cp.wait()
        wbf_ref[...] = wraw_ref[...].T.astype(bf16)

    mem = mem_ref[...]
    memb = mem.astype(bf16)
    cin = jnp.concatenate(
        [x_ref[...].astype(bf16), h0_ref[...].astype(bf16), memb], axis=1)
    acc = jax.lax.dot_general(cin, wbf_ref[...], dnn, preferred_element_type=f32)
    h = jnp.maximum(acc + bh_ref[...], 0.0)
    h_ref[...] = h

    hm = jnp.concatenate([h.astype(bf16), memb], axis=1)
    u = jax.lax.dot_general(hm, wct_ref[...], dnn, preferred_element_type=f32)
    u += bc_ref[...]

    uv = u[:, :128]
    ab = u[:, _A0:_A0 + 2 * K]
    u2 = uv * uv
    p = u2 * u2 * jnp.abs(uv)
    s = jnp.dot(p, ms_ref[...], preferred_element_type=f32)
    n5 = jnp.dot(s, mn0_ref[...], preferred_element_type=f32) \
        * jnp.dot(s, mn1_ref[...], preferred_element_type=f32)
    n = jnp.exp2(0.2 * jnp.log2(n5))
    coef = ab * (0.25 / jnp.maximum(n, EPS))
    sel = jnp.dot(uv, mcomb_ref[...], preferred_element_type=f32)
    cexp = jnp.dot(coef, mcexp_ref[...], preferred_element_type=f32)
    g = cexp * sel[:, :64]
    gf = jnp.dot(g, mfold_ref[...], preferred_element_type=f32)
    gexp = jnp.dot(gf, mgexp_ref[...], preferred_element_type=f32)
    prod = gexp * sel[:, 128:640]
    memnew_ref[...] = mem + (prod[:, :MEM] - prod[:, MEM:])


def kernel(x, h0, memory, W_h, b_h, W_a, b_a, W_b, b_b, W_va, b_va, W_vb, b_vb):
    zpad_w = jnp.zeros((NUP - 4 * SMK - 2 * K, HID + MEM), jnp.float32)
    wct = jnp.concatenate([W_va, W_vb, W_a, W_b, zpad_w],
                          axis=0).T.astype(jnp.bfloat16)
    zpad_b = jnp.zeros((NUP - 4 * SMK - 2 * K,), jnp.float32)
    bc = jnp.concatenate([b_va, b_vb, b_a, b_b, zpad_b])[None, :]
    bh = b_h[None, :]

    consts = [jnp.asarray(c) for c in _CONSTS]

    grid = (B // BLK,)
    row_spec = lambda cols: pl.BlockSpec((BLK, cols), lambda i: (i, 0))
    full = lambda a: pl.BlockSpec(a.shape, lambda i: (0,) * a.ndim)

    memnew, h = pl.pallas_call(
        _nru_kernel,
        grid=grid,
        in_specs=[row_spec(INP), row_spec(HID), row_spec(MEM),
                  pl.BlockSpec(memory_space=pl.ANY), full(bh),
                  full(wct), full(bc)] + [full(c) for c in consts],
        out_specs=[row_spec(MEM), row_spec(HID)],
        out_shape=[jax.ShapeDtypeStruct((B, MEM), jnp.float32),
                   jax.ShapeDtypeStruct((B, HID), jnp.float32)],
        scratch_shapes=[pltpu.VMEM((INP + HID + MEM, HID), jnp.bfloat16),
                        pltpu.VMEM((HID, INP + HID + MEM), jnp.float32),
                        pltpu.SemaphoreType.DMA],
        compiler_params=pltpu.CompilerParams(
            dimension_semantics=("arbitrary",),
            vmem_limit_bytes=100 * 1024 * 1024,
        ),
    )(x, h0, memory, W_h, bh, wct, bc, *consts)
    return memnew, h


# speedup vs baseline: 1.1460x; 1.1175x over previous
import functools

import jax
import jax.numpy as jnp
import numpy as np
from jax.experimental import pallas as pl
from jax.experimental.pallas import tpu as pltpu

MEM = 256
K = 4
SMK = 32
HID = 1024
INP = 1024
B = 8192
EPS = 1e-12
BLK = 1024

NUP = 256
_U0A, _U1A, _U0B, _U1B, _A0, _B0 = 0, SMK, 2 * SMK, 3 * SMK, 4 * SMK, 4 * SMK + K


def _build_consts():
    ms = np.zeros((128, 16), np.float32)
    for i in range(SMK):
        ms[_U0A + i, i // 8] = 1.0
        ms[_U0B + i, 4 + i // 8] = 1.0
        ms[_U1A + i, 8] = 1.0
        ms[_U1B + i, 9] = 1.0
    mn0 = np.zeros((16, 8), np.float32)
    mn1 = np.zeros((16, 8), np.float32)
    for k in range(K):
        mn0[k, k] = 1.0
        mn0[4 + k, 4 + k] = 1.0
        mn1[8, k] = 1.0
        mn1[9, 4 + k] = 1.0
    mcexp = np.zeros((8, 64), np.float32)
    for i in range(SMK):
        mcexp[i // 8, i] = 1.0
        mcexp[4 + i // 8, 32 + i] = 1.0
    mcomb = np.zeros((128, 640), np.float32)
    for i in range(SMK):
        mcomb[_U0A + i, i] = 1.0
        mcomb[_U0B + i, 32 + i] = 1.0
    for m in range(MEM):
        mcomb[_U1A + m % 32, 128 + m] = 1.0
        mcomb[_U1B + m % 32, 384 + m] = 1.0
    mfold = np.zeros((64, 16), np.float32)
    for i in range(SMK):
        mfold[i, i % 8] = 1.0
        mfold[32 + i, 8 + i % 8] = 1.0
    mgexp = np.zeros((16, 512), np.float32)
    for m in range(MEM):
        mgexp[m // 32, m] = 1.0
        mgexp[8 + m // 32, 256 + m] = 1.0
    return ms, mn0, mn1, mcexp, mcomb, mfold, mgexp


_CONSTS = _build_consts()


def _nru_kernel(x_ref, h0_ref, mem_ref, wh_ref, bh_ref,
                wva_ref, wvb_ref, wa_ref, wb_ref,
                bva_ref, bvb_ref, ba_ref, bb_ref,
                ms_ref, mn0_ref, mn1_ref, mcexp_ref, mcomb_ref,
                mfold_ref, mgexp_ref,
                memnew_ref, h_ref, wbf_ref, wraw_ref, wcat_scr, wct_ref,
                bc_ref, wsem):
    f32 = jnp.float32
    bf16 = jnp.bfloat16
    dnn = (((1,), (0,)), ((), ()))

    @pl.when(pl.program_id(0) == 0)
    def _cast_weights():
        cp = pltpu.make_async_copy(wh_ref, wraw_ref, wsem)
        cp.start()
        wcat_scr[...] = jnp.zeros_like(wcat_scr)
        wcat_scr[0:2 * SMK, :] = wva_ref[...]
        wcat_scr[2 * SMK:4 * SMK, :] = wvb_ref[...]
        wcat_scr[_A0:_A0 + K, :] = wa_ref[...]
        wcat_scr[_B0:_B0 + K, :] = wb_ref[...]
        wct_ref[...] = wcat_scr[...].T.astype(bf16)
        bc_ref[...] = jnp.zeros_like(bc_ref)
        bc_ref[:, 0:2 * SMK] = bva_ref[...]
        bc_ref[:, 2 * SMK:4 * SMK] = bvb_ref[...]
        bc_ref[:, _A0:_A0 + K] = ba_ref[...]
        bc_ref[:, _B0:_B0 + K] = bb_ref[...]
        cp.wait()
        wbf_ref[...] = wraw_ref[...].T.astype(bf16)

    mem = mem_ref[...]
    memb = mem.astype(bf16)
    cin = jnp.concatenate(
        [x_ref[...].astype(bf16), h0_ref[...].astype(bf16), memb], axis=1)
    acc = jax.lax.dot_general(cin, wbf_ref[...], dnn, preferred_element_type=f32)
    h = jnp.maximum(acc + bh_ref[...], 0.0)
    h_ref[...] = h

    hm = jnp.concatenate([h.astype(bf16), memb], axis=1)
    u = jax.lax.dot_general(hm, wct_ref[...], dnn, preferred_element_type=f32)
    u += bc_ref[...]

    uv = u[:, :128]
    ab = u[:, _A0:_A0 + 2 * K]
    u2 = uv * uv
    p = u2 * u2 * jnp.abs(uv)
    s = jnp.dot(p, ms_ref[...], preferred_element_type=f32)
    n5 = jnp.dot(s, mn0_ref[...], preferred_element_type=f32) \
        * jnp.dot(s, mn1_ref[...], preferred_element_type=f32)
    n = jnp.exp2(0.2 * jnp.log2(n5))
    coef = ab * (0.25 / jnp.maximum(n, EPS))
    sel = jnp.dot(uv, mcomb_ref[...], preferred_element_type=f32)
    cexp = jnp.dot(coef, mcexp_ref[...], preferred_element_type=f32)
    g = cexp * sel[:, :64]
    gf = jnp.dot(g, mfold_ref[...], preferred_element_type=f32)
    gexp = jnp.dot(gf, mgexp_ref[...], preferred_element_type=f32)
    prod = gexp * sel[:, 128:640]
    memnew_ref[...] = mem + (prod[:, :MEM] - prod[:, MEM:])


def kernel(x, h0, memory, W_h, b_h, W_a, b_a, W_b, b_b, W_va, b_va, W_vb, b_vb):
    bh = b_h[None, :]

    consts = [jnp.asarray(c) for c in _CONSTS]

    grid = (B // BLK,)
    row_spec = lambda cols: pl.BlockSpec((BLK, cols), lambda i: (i, 0))
    full = lambda a: pl.BlockSpec(a.shape, lambda i: (0,) * a.ndim)

    memnew, h = pl.pallas_call(
        _nru_kernel,
        grid=grid,
        in_specs=[row_spec(INP), row_spec(HID), row_spec(MEM),
                  pl.BlockSpec(memory_space=pl.ANY), full(bh),
                  full(W_va), full(W_vb), full(W_a), full(W_b),
                  full(b_va[None, :]), full(b_vb[None, :]),
                  full(b_a[None, :]), full(b_b[None, :])]
                 + [full(c) for c in consts],
        out_specs=[row_spec(MEM), row_spec(HID)],
        out_shape=[jax.ShapeDtypeStruct((B, MEM), jnp.float32),
                   jax.ShapeDtypeStruct((B, HID), jnp.float32)],
        scratch_shapes=[pltpu.VMEM((INP + HID + MEM, HID), jnp.bfloat16),
                        pltpu.VMEM((HID, INP + HID + MEM), jnp.float32),
                        pltpu.VMEM((NUP, HID + MEM), jnp.float32),
                        pltpu.VMEM((HID + MEM, NUP), jnp.bfloat16),
                        pltpu.VMEM((1, NUP), jnp.float32),
                        pltpu.SemaphoreType.DMA],
        compiler_params=pltpu.CompilerParams(
            dimension_semantics=("arbitrary",),
            vmem_limit_bytes=100 * 1024 * 1024,
        ),
    )(x, h0, memory, W_h, bh, W_va, W_vb, W_a, W_b,
      b_va[None, :], b_vb[None, :], b_a[None, :], b_b[None, :], *consts)
    return memnew, h
